# baseline (speedup 1.0000x reference)
import math

import jax
import jax.numpy as jnp
from jax.experimental import pallas as pl
from jax.experimental.pallas import tpu as pltpu

_GOLDEN_CENTER = 1 / math.e
_GOLDEN_LOWER = 0.5 - math.log(4 / 3)
_GOLDEN_UPPER = 0.5

_NF = 4


def _moe_body(temp_ref, x_ref, wr_ref, wg_ref, wu_ref, wd_ref, out_ref,
              xb_s, hb_s, w_s):
    k = pl.program_id(0)
    n_slices = pl.num_programs(0) - 1
    E = wr_ref.shape[0]
    dn = (((1,), (1,)), ((), ()))

    @pl.when(k == 0)
    def _router_and_xcast():
        xc = x_ref[...]
        xb_s[...] = xc.astype(jnp.bfloat16)
        logits = jax.lax.dot_general(
            xc, wr_ref[...], dn,
            preferred_element_type=jnp.float32)
        inhib = jax.nn.sigmoid(logits / temp_ref[0])
        in_zone = jnp.logical_and(inhib >= _GOLDEN_LOWER, inhib <= _GOLDEN_UPPER)
        dist = jnp.abs(inhib - _GOLDEN_CENTER)
        w = jnp.exp(-dist / 0.1) * in_zone.astype(jnp.float32)
        wsum = jnp.sum(w, axis=1, keepdims=True)
        no_expert = wsum < 1e-8
        fb = jnp.exp(-dist / 0.3)
        idx = jax.lax.broadcasted_iota(jnp.int32, fb.shape, 1)
        m1 = jnp.max(fb, axis=1, keepdims=True)
        i1 = jnp.min(jnp.where(fb == m1, idx, E), axis=1, keepdims=True)
        mask1 = idx == i1
        fb2 = jnp.where(mask1, -jnp.inf, fb)
        m2 = jnp.max(fb2, axis=1, keepdims=True)
        i2 = jnp.min(jnp.where(fb2 == m2, idx, E), axis=1, keepdims=True)
        fbm = jnp.logical_or(mask1, idx == i2).astype(jnp.float32)
        fb_w = fb * fbm
        fb_w = fb_w / jnp.maximum(jnp.sum(fb_w, axis=1, keepdims=True), 1e-8)
        w = jnp.where(no_expert, fb_w, w)
        w = w / jnp.maximum(jnp.sum(w, axis=1, keepdims=True), 1e-8)
        w_s[...] = w

    @pl.when(k > 0)
    def _down_prev():
        e_prev = (k - 1) // _NF
        hprev = hb_s[(k - 1) % 2]
        d = jax.lax.dot_general(hprev, wd_ref[0].astype(jnp.bfloat16), dn,
                                preferred_element_type=jnp.float32)
        wc = w_s[...]
        onehot = (jax.lax.broadcasted_iota(jnp.int32, wc.shape, 1) == e_prev)
        wcol = jnp.sum(wc * onehot.astype(jnp.float32), axis=1, keepdims=True)
        contrib = d * wcol

        @pl.when(k == 1)
        def _init():
            out_ref[...] = contrib

        @pl.when(k > 1)
        def _accum():
            out_ref[...] += contrib

    @pl.when(k < n_slices)
    def _gate_up():
        xb = xb_s[...]
        gate = jax.lax.dot_general(xb, wg_ref[0].astype(jnp.bfloat16), dn,
                                   preferred_element_type=jnp.float32)
        up = jax.lax.dot_general(xb, wu_ref[0].astype(jnp.bfloat16), dn,
                                 preferred_element_type=jnp.float32)
        h = (gate * jax.nn.sigmoid(gate)) * up
        hb_s[k % 2] = h.astype(jnp.bfloat16)


def kernel(x, Wr, Wg, Wu, Wd, temperature):
    B, T, D = x.shape
    E, F, _ = Wg.shape
    FS = F // _NF
    n_slices = E * _NF
    x2 = x.reshape(B * T, D)

    def _gu_idx(k):
        kc = jnp.minimum(k, n_slices - 1)
        return (kc // _NF, kc % _NF, 0)

    def _wd_idx(k):
        km1 = jnp.maximum(k - 1, 0)
        return (km1 // _NF, 0, km1 % _NF)

    out = pl.pallas_call(
        _moe_body,
        grid=(n_slices + 1,),
        in_specs=[
            pl.BlockSpec(memory_space=pltpu.SMEM),
            pl.BlockSpec((B * T, D), lambda k: (0, 0)),
            pl.BlockSpec((E, D), lambda k: (0, 0)),
            pl.BlockSpec((1, FS, D), _gu_idx),
            pl.BlockSpec((1, FS, D), _gu_idx),
            pl.BlockSpec((1, D, FS), _wd_idx),
        ],
        out_specs=pl.BlockSpec((B * T, D), lambda k: (0, 0)),
        out_shape=jax.ShapeDtypeStruct((B * T, D), jnp.float32),
        scratch_shapes=[
            pltpu.VMEM((B * T, D), jnp.bfloat16),
            pltpu.VMEM((2, B * T, FS), jnp.bfloat16),
            pltpu.VMEM((B * T, E), jnp.float32),
        ],
    )(temperature, x2, Wr, Wg, Wu, Wd)
    return out.reshape(B, T, D).astype(x.dtype)


# speedup vs baseline: 1.1246x; 1.1246x over previous
import math

import jax
import jax.numpy as jnp
from jax.experimental import pallas as pl
from jax.experimental.pallas import tpu as pltpu

_GOLDEN_CENTER = 1 / math.e
_GOLDEN_LOWER = 0.5 - math.log(4 / 3)
_GOLDEN_UPPER = 0.5

_NF = 4


def _moe_body(temp_ref, x_ref, wr_ref, wg_ref, wu_ref, wd_ref, out_ref,
              xb_s, hb_s, w_s):
    k = pl.program_id(0)
    n_slices = pl.num_programs(0) - 1
    E = wr_ref.shape[0]
    dn = (((1,), (1,)), ((), ()))

    @pl.when(k == 0)
    def _router_and_xcast():
        xc = x_ref[...]
        xb_s[...] = xc.astype(jnp.bfloat16)
        logits = jax.lax.dot_general(
            xc, wr_ref[...], dn,
            preferred_element_type=jnp.float32)
        inhib = jax.nn.sigmoid(logits / temp_ref[0])
        in_zone = jnp.logical_and(inhib >= _GOLDEN_LOWER, inhib <= _GOLDEN_UPPER)
        dist = jnp.abs(inhib - _GOLDEN_CENTER)
        w = jnp.exp(-dist / 0.1) * in_zone.astype(jnp.float32)
        wsum = jnp.sum(w, axis=1, keepdims=True)
        no_expert = wsum < 1e-8
        fb = jnp.exp(-dist / 0.3)
        idx = jax.lax.broadcasted_iota(jnp.int32, fb.shape, 1)
        m1 = jnp.max(fb, axis=1, keepdims=True)
        i1 = jnp.min(jnp.where(fb == m1, idx, E), axis=1, keepdims=True)
        mask1 = idx == i1
        fb2 = jnp.where(mask1, -jnp.inf, fb)
        m2 = jnp.max(fb2, axis=1, keepdims=True)
        i2 = jnp.min(jnp.where(fb2 == m2, idx, E), axis=1, keepdims=True)
        fbm = jnp.logical_or(mask1, idx == i2).astype(jnp.float32)
        fb_w = fb * fbm
        fb_w = fb_w / jnp.maximum(jnp.sum(fb_w, axis=1, keepdims=True), 1e-8)
        w = jnp.where(no_expert, fb_w, w)
        w = w / jnp.maximum(jnp.sum(w, axis=1, keepdims=True), 1e-8)
        w_s[...] = w

    xb = xb_s[...]
    gate = jax.lax.dot_general(xb, wg_ref[0].astype(jnp.bfloat16), dn,
                               preferred_element_type=jnp.float32)
    up = jax.lax.dot_general(xb, wu_ref[0].astype(jnp.bfloat16), dn,
                             preferred_element_type=jnp.float32)
    h = (gate * jax.nn.sigmoid(gate)) * up

    e_prev = (k - 1) // _NF
    hprev = hb_s[(k - 1) % 2]
    d = jax.lax.dot_general(hprev, wd_ref[0].astype(jnp.bfloat16), dn,
                            preferred_element_type=jnp.float32)
    wc = w_s[...]
    onehot = (jax.lax.broadcasted_iota(jnp.int32, wc.shape, 1) == e_prev)
    wcol = jnp.sum(wc * onehot.astype(jnp.float32), axis=1, keepdims=True)
    contrib = jnp.where(k == 0, 0.0, d * wcol)
    out_ref[...] = jnp.where(k <= 1, 0.0, out_ref[...]) + contrib

    hb_s[k % 2] = h.astype(jnp.bfloat16)


def kernel(x, Wr, Wg, Wu, Wd, temperature):
    B, T, D = x.shape
    E, F, _ = Wg.shape
    FS = F // _NF
    n_slices = E * _NF
    x2 = x.reshape(B * T, D)

    def _gu_idx(k):
        kc = jnp.minimum(k, n_slices - 1)
        return (kc // _NF, kc % _NF, 0)

    def _wd_idx(k):
        km1 = jnp.maximum(k - 1, 0)
        return (km1 // _NF, 0, km1 % _NF)

    out = pl.pallas_call(
        _moe_body,
        grid=(n_slices + 1,),
        in_specs=[
            pl.BlockSpec(memory_space=pltpu.SMEM),
            pl.BlockSpec((B * T, D), lambda k: (0, 0)),
            pl.BlockSpec((E, D), lambda k: (0, 0)),
            pl.BlockSpec((1, FS, D), _gu_idx),
            pl.BlockSpec((1, FS, D), _gu_idx),
            pl.BlockSpec((1, D, FS), _wd_idx),
        ],
        out_specs=pl.BlockSpec((B * T, D), lambda k: (0, 0)),
        out_shape=jax.ShapeDtypeStruct((B * T, D), jnp.float32),
        scratch_shapes=[
            pltpu.VMEM((B * T, D), jnp.bfloat16),
            pltpu.VMEM((2, B * T, FS), jnp.bfloat16),
            pltpu.VMEM((B * T, E), jnp.float32),
        ],
    )(temperature, x2, Wr, Wg, Wu, Wd)
    return out.reshape(B, T, D).astype(x.dtype)
